# baseline (speedup 1.0000x reference)
import functools

import jax
import jax.numpy as jnp
from jax import lax
from jax.experimental import pallas as pl
from jax.experimental.pallas import tpu as pltpu
from jax.experimental.pallas import tpu_sc as plsc

N = 10000
E = 320000
D = 128
H = 128
NUM_LEVELS = 3
NUM_PROC = 9

NC = 2
NS = 16
NW = NC * NS
CH = 128
CPW = 81
EPAD = NW * CPW * CH
NP = 10112
ROWS_PER_TILE = NP // NS
NB = 3


def _sc_body(x_hbm, src_hbm, dst_hbm, out_hbm,
             sidx0, sidx1, sidx2, didx0, didx1, didx2,
             rows0, rows1, rows2, aggr_sh,
             semr0, semr1, semr2, semd0, semd1, semd2,
             sems0, sems1, sems2, semsc0, semsc1, semsc2):
    c = lax.axis_index("c")
    s = lax.axis_index("s")
    wid = s * NC + c

    def zrow(r, carry):
        for cc in range(D // 16):
            rows0[r, pl.ds(cc * 16, 16)] = jnp.zeros((16,), jnp.float32)
        return carry

    lax.fori_loop(0, CH, zrow, 0)
    for t in range(ROWS_PER_TILE // CH):
        pltpu.sync_copy(
            rows0, aggr_sh.at[pl.ds(s * ROWS_PER_TILE + t * CH, CH)])
    rem = ROWS_PER_TILE % CH
    if rem:
        pltpu.sync_copy(
            rows0.at[pl.ds(0, rem)],
            aggr_sh.at[pl.ds(s * ROWS_PER_TILE + ROWS_PER_TILE - rem, rem)])
    plsc.subcore_barrier()

    ebase = wid * CPW * CH
    rows = (rows0, rows1, rows2)
    sidx = (sidx0, sidx1, sidx2)
    didx = (didx0, didx1, didx2)
    semr = (semr0, semr1, semr2)
    semd = (semd0, semd1, semd2)
    sems = (sems0, sems1, sems2)
    semsc = (semsc0, semsc1, semsc2)

    pltpu.async_copy(src_hbm.at[pl.ds(ebase, CH)], sidx0, sems0)
    pltpu.async_copy(src_hbm.at[pl.ds(ebase + CH, CH)], sidx1, sems1)
    pltpu.async_copy(dst_hbm.at[pl.ds(ebase, CH)], didx0, semd0)
    pltpu.make_async_copy(src_hbm.at[pl.ds(ebase, CH)], sidx0, sems0).wait()
    pltpu.async_copy(x_hbm.at[sidx0], rows0, semr0)

    def step(i, carry):
        for b in range(NB):
            kk = NB * i + b
            n1 = (b + 1) % NB
            n2 = (b + 2) % NB

            @pl.when(kk + 1 < CPW)
            def _():
                pltpu.make_async_copy(
                    src_hbm.at[pl.ds(ebase + (kk + 1) * CH, CH)],
                    sidx[n1], sems[n1]).wait()

                @pl.when(kk >= 2)
                def _():
                    pltpu.make_async_copy(rows[n1], aggr_sh.at[didx[n1]],
                                          semsc[n1]).wait()
                pltpu.async_copy(
                    dst_hbm.at[pl.ds(ebase + (kk + 1) * CH, CH)],
                    didx[n1], semd[n1])
                pltpu.async_copy(x_hbm.at[sidx[n1]], rows[n1], semr[n1])

            pltpu.make_async_copy(dst_hbm.at[pl.ds(ebase + kk * CH, CH)],
                                  didx[b], semd[b]).wait()
            pltpu.make_async_copy(x_hbm.at[sidx[b]], rows[b], semr[b]).wait()
            pltpu.async_copy(rows[b], aggr_sh.at[didx[b]], semsc[b], add=True)

            @pl.when(kk + 2 < CPW)
            def _():
                pltpu.async_copy(src_hbm.at[pl.ds(ebase + (kk + 2) * CH, CH)],
                                 sidx[n2], sems[n2])
        return carry

    lax.fori_loop(0, CPW // NB, step, 0)

    for b in range(NB):
        pltpu.make_async_copy(rows[b], aggr_sh.at[didx[b]], semsc[b]).wait()

    plsc.subcore_barrier()
    obase = s * ROWS_PER_TILE
    for t in range(ROWS_PER_TILE // CH):
        pltpu.sync_copy(aggr_sh.at[pl.ds(obase + t * CH, CH)],
                        out_hbm.at[c, pl.ds(obase + t * CH, CH)])
    if rem:
        pltpu.sync_copy(
            aggr_sh.at[pl.ds(obase + ROWS_PER_TILE - rem, rem)],
            out_hbm.at[c, pl.ds(obase + ROWS_PER_TILE - rem, rem)])


@functools.cache
def _sc_segment_sum():
    return pl.kernel(
        _sc_body,
        out_type=jax.ShapeDtypeStruct((NC, NP, D), jnp.float32),
        mesh=plsc.VectorSubcoreMesh(core_axis_name="c", subcore_axis_name="s",
                                    num_cores=NC, num_subcores=NS),
        scratch_types=(
            [pltpu.VMEM((CH,), jnp.int32) for _ in range(2 * NB)]
            + [pltpu.VMEM((CH, D), jnp.float32) for _ in range(NB)]
            + [pltpu.VMEM_SHARED((NP, D), jnp.float32)]
            + [pltpu.SemaphoreType.DMA for _ in range(4 * NB)]
        ),
    )


ROW_BLK = 2000


def _tc_body(x_ref, parts_ref, w_ref, b_ref, s_ref, *out_refs):
    xa = jnp.concatenate([x_ref[...], parts_ref[0] + parts_ref[1]], axis=1)
    for level in range(NUM_LEVELS):
        xc = (jnp.dot(xa, w_ref[level], preferred_element_type=jnp.float32)
              + b_ref[level][None, :])
        big = jnp.dot(xc, s_ref[level], preferred_element_type=jnp.float32)
        for j in range(3):
            out_refs[level * 3 + j][...] = big[:, j * H:(j + 1) * H]


def _tc_dense(x, parts, Wstack, b, Sstack):
    grid = (N // ROW_BLK,)
    full = lambda shape: pl.BlockSpec(shape, lambda i: (0,) * len(shape))
    return pl.pallas_call(
        _tc_body,
        grid=grid,
        in_specs=[
            pl.BlockSpec((ROW_BLK, D), lambda i: (i, 0)),
            pl.BlockSpec((NC, ROW_BLK, D), lambda i: (0, i, 0)),
            full((NUM_LEVELS, 2 * D, H)),
            full((NUM_LEVELS, H)),
            full((NUM_LEVELS, H, 3 * H)),
        ],
        out_specs=tuple(pl.BlockSpec((ROW_BLK, H), lambda i: (i, 0))
                        for _ in range(NUM_PROC)),
        out_shape=tuple(jax.ShapeDtypeStruct((N, H), jnp.float32)
                        for _ in range(NUM_PROC)),
    )(x, parts, Wstack, b, Sstack)


def kernel(x, edge_index, Wroot, Wrel, b, S):
    pad = EPAD - E
    pad_ar = jnp.arange(pad, dtype=jnp.int32)
    src = jnp.concatenate([edge_index[0], pad_ar % N])
    dst = jnp.concatenate([edge_index[1], N + (pad_ar % (NP - N))])
    parts = _sc_segment_sum()(x, src, dst)
    Wstack = jnp.concatenate([Wroot, Wrel], axis=1)
    Sstack = jnp.transpose(S.reshape(NUM_LEVELS, 3, H, H),
                           (0, 2, 1, 3)).reshape(NUM_LEVELS, H, 3 * H)
    outs = _tc_dense(x, parts, Wstack, b, Sstack)
    return tuple(outs)


# speedup vs baseline: 1.0166x; 1.0166x over previous
import functools

import jax
import jax.numpy as jnp
from jax import lax
from jax.experimental import pallas as pl
from jax.experimental.pallas import tpu as pltpu
from jax.experimental.pallas import tpu_sc as plsc

N = 10000
E = 320000
D = 128
H = 128
NUM_LEVELS = 3
NUM_PROC = 9

NC = 2
NS = 16
NW = NC * NS
CH = 128
CPW = 80
EPAD = NW * CPW * CH
NP = 10240
ROWS_PER_TILE = NP // NS


def _sc_body(x_hbm, src_hbm, dst_hbm, out_hbm,
             sidx0, sidx1, didx0, didx1, rows0, rows1, aggr_sh,
             semr0, semr1, semd0, semd1, sems0, sems1, semsc0, semsc1):
    c = lax.axis_index("c")
    s = lax.axis_index("s")
    wid = s * NC + c

    def zrow(r, carry):
        for cc in range(D // 16):
            rows0[r, pl.ds(cc * 16, 16)] = jnp.zeros((16,), jnp.float32)
        return carry

    lax.fori_loop(0, CH, zrow, 0)
    for t in range(ROWS_PER_TILE // CH):
        pltpu.sync_copy(
            rows0, aggr_sh.at[pl.ds(s * ROWS_PER_TILE + t * CH, CH)])
    plsc.subcore_barrier()

    ebase = wid * CPW * CH
    rows = (rows0, rows1)
    sidx = (sidx0, sidx1)
    didx = (didx0, didx1)
    semr = (semr0, semr1)
    semd = (semd0, semd1)
    sems = (sems0, sems1)
    semsc = (semsc0, semsc1)

    pltpu.async_copy(src_hbm.at[pl.ds(ebase, CH)], sidx0, sems0)
    pltpu.async_copy(src_hbm.at[pl.ds(ebase + CH, CH)], sidx1, sems1)
    pltpu.async_copy(dst_hbm.at[pl.ds(ebase, CH)], didx0, semd0)
    pltpu.make_async_copy(src_hbm.at[pl.ds(ebase, CH)], sidx0, sems0).wait()
    pltpu.async_copy(x_hbm.at[sidx0], rows0, semr0)

    def step(i, carry):
        for b in range(2):
            kk = 2 * i + b
            nb = b ^ 1

            @pl.when(kk + 1 < CPW)
            def _():
                pltpu.make_async_copy(
                    src_hbm.at[pl.ds(ebase + (kk + 1) * CH, CH)],
                    sidx[nb], sems[nb]).wait()

                @pl.when(kk >= 1)
                def _():
                    pltpu.make_async_copy(rows[nb], aggr_sh.at[didx[nb]],
                                          semsc[nb]).wait()
                pltpu.async_copy(
                    dst_hbm.at[pl.ds(ebase + (kk + 1) * CH, CH)],
                    didx[nb], semd[nb])
                pltpu.async_copy(x_hbm.at[sidx[nb]], rows[nb], semr[nb])

            pltpu.make_async_copy(dst_hbm.at[pl.ds(ebase + kk * CH, CH)],
                                  didx[b], semd[b]).wait()
            pltpu.make_async_copy(x_hbm.at[sidx[b]], rows[b], semr[b]).wait()
            pltpu.async_copy(rows[b], aggr_sh.at[didx[b]], semsc[b], add=True)

            @pl.when(kk + 2 < CPW)
            def _():
                pltpu.async_copy(src_hbm.at[pl.ds(ebase + (kk + 2) * CH, CH)],
                                 sidx[b], sems[b])
        return carry

    lax.fori_loop(0, CPW // 2, step, 0)

    pltpu.make_async_copy(rows[0], aggr_sh.at[didx[0]], semsc[0]).wait()
    pltpu.make_async_copy(rows[1], aggr_sh.at[didx[1]], semsc[1]).wait()

    plsc.subcore_barrier()
    pltpu.sync_copy(aggr_sh.at[pl.ds(s * ROWS_PER_TILE, ROWS_PER_TILE)],
                    out_hbm.at[c, pl.ds(s * ROWS_PER_TILE, ROWS_PER_TILE)])


@functools.cache
def _sc_segment_sum():
    return pl.kernel(
        _sc_body,
        out_type=jax.ShapeDtypeStruct((NC, NP, D), jnp.float32),
        mesh=plsc.VectorSubcoreMesh(core_axis_name="c", subcore_axis_name="s",
                                    num_cores=NC, num_subcores=NS),
        scratch_types=[
            pltpu.VMEM((CH,), jnp.int32),
            pltpu.VMEM((CH,), jnp.int32),
            pltpu.VMEM((CH,), jnp.int32),
            pltpu.VMEM((CH,), jnp.int32),
            pltpu.VMEM((CH, D), jnp.float32),
            pltpu.VMEM((CH, D), jnp.float32),
            pltpu.VMEM_SHARED((NP, D), jnp.float32),
            pltpu.SemaphoreType.DMA,
            pltpu.SemaphoreType.DMA,
            pltpu.SemaphoreType.DMA,
            pltpu.SemaphoreType.DMA,
            pltpu.SemaphoreType.DMA,
            pltpu.SemaphoreType.DMA,
            pltpu.SemaphoreType.DMA,
            pltpu.SemaphoreType.DMA,
        ],
    )


ROW_BLK = 2000


def _tc_body(x_ref, parts_ref, w_ref, b_ref, s_ref, *out_refs):
    xa = jnp.concatenate([x_ref[...], parts_ref[0] + parts_ref[1]], axis=1)
    xa = xa.astype(jnp.bfloat16)
    for level in range(NUM_LEVELS):
        xc = (jnp.dot(xa, w_ref[level].astype(jnp.bfloat16),
                      preferred_element_type=jnp.float32)
              + b_ref[level][None, :])
        big = jnp.dot(xc.astype(jnp.bfloat16),
                      s_ref[level].astype(jnp.bfloat16),
                      preferred_element_type=jnp.float32)
        for j in range(3):
            out_refs[level * 3 + j][...] = big[:, j * H:(j + 1) * H]


def _tc_dense(x, parts, Wstack, b, Sstack):
    grid = (N // ROW_BLK,)
    full = lambda shape: pl.BlockSpec(shape, lambda i: (0,) * len(shape))
    return pl.pallas_call(
        _tc_body,
        grid=grid,
        in_specs=[
            pl.BlockSpec((ROW_BLK, D), lambda i: (i, 0)),
            pl.BlockSpec((NC, ROW_BLK, D), lambda i: (0, i, 0)),
            full((NUM_LEVELS, 2 * D, H)),
            full((NUM_LEVELS, H)),
            full((NUM_LEVELS, H, 3 * H)),
        ],
        out_specs=tuple(pl.BlockSpec((ROW_BLK, H), lambda i: (i, 0))
                        for _ in range(NUM_PROC)),
        out_shape=tuple(jax.ShapeDtypeStruct((N, H), jnp.float32)
                        for _ in range(NUM_PROC)),
    )(x, parts, Wstack, b, Sstack)


def kernel(x, edge_index, Wroot, Wrel, b, S):
    pad = EPAD - E
    pad_ar = jnp.arange(pad, dtype=jnp.int32)
    src = jnp.concatenate([edge_index[0], pad_ar % N])
    dst = jnp.concatenate([edge_index[1], N + (pad_ar % (NP - N))])
    parts = _sc_segment_sum()(x, src, dst)
    Wstack = jnp.concatenate([Wroot, Wrel], axis=1)
    Sstack = jnp.transpose(S.reshape(NUM_LEVELS, 3, H, H),
                           (0, 2, 1, 3)).reshape(NUM_LEVELS, H, 3 * H)
    outs = _tc_dense(x, parts, Wstack, b, Sstack)
    return tuple(outs)


# speedup vs baseline: 1.0193x; 1.0026x over previous
import functools

import jax
import jax.numpy as jnp
from jax import lax
from jax.experimental import pallas as pl
from jax.experimental.pallas import tpu as pltpu
from jax.experimental.pallas import tpu_sc as plsc

N = 10000
E = 320000
D = 128
H = 128
NUM_LEVELS = 3
NUM_PROC = 9

NC = 2
NS = 16
NW = NC * NS
CH = 128
CPW = 80
EPAD = NW * CPW * CH
NP = 10240
ROWS_PER_TILE = NP // NS


def _sc_body(x_hbm, src_hbm, dst_hbm, out_hbm,
             sidx0, sidx1, didx0, didx1, rows0, rows1, aggr_sh,
             semr0, semr1, semd0, semd1, sems0, sems1, semsc0, semsc1):
    c = lax.axis_index("c")
    s = lax.axis_index("s")
    wid = s * NC + c

    def zrow(r, carry):
        for cc in range(D // 16):
            rows0[r, pl.ds(cc * 16, 16)] = jnp.zeros((16,), jnp.float32)
        return carry

    lax.fori_loop(0, CH, zrow, 0)
    for t in range(ROWS_PER_TILE // CH):
        pltpu.sync_copy(
            rows0, aggr_sh.at[pl.ds(s * ROWS_PER_TILE + t * CH, CH)])
    plsc.subcore_barrier()

    ebase = wid * CPW * CH
    rows = (rows0, rows1)
    sidx = (sidx0, sidx1)
    didx = (didx0, didx1)
    semr = (semr0, semr1)
    semd = (semd0, semd1)
    sems = (sems0, sems1)
    semsc = (semsc0, semsc1)

    pltpu.async_copy(src_hbm.at[pl.ds(ebase, CH)], sidx0, sems0)
    pltpu.async_copy(src_hbm.at[pl.ds(ebase + CH, CH)], sidx1, sems1)
    pltpu.async_copy(dst_hbm.at[pl.ds(ebase, CH)], didx0, semd0)
    pltpu.make_async_copy(src_hbm.at[pl.ds(ebase, CH)], sidx0, sems0).wait()
    pltpu.async_copy(x_hbm.at[sidx0], rows0, semr0)

    def step(i, carry):
        for b in range(2):
            kk = 2 * i + b
            nb = b ^ 1

            @pl.when(kk + 1 < CPW)
            def _():
                pltpu.make_async_copy(
                    src_hbm.at[pl.ds(ebase + (kk + 1) * CH, CH)],
                    sidx[nb], sems[nb]).wait()

                @pl.when(kk >= 1)
                def _():
                    pltpu.make_async_copy(rows[nb], aggr_sh.at[didx[nb]],
                                          semsc[nb]).wait()
                pltpu.async_copy(
                    dst_hbm.at[pl.ds(ebase + (kk + 1) * CH, CH)],
                    didx[nb], semd[nb])
                pltpu.async_copy(x_hbm.at[sidx[nb]], rows[nb], semr[nb])

            pltpu.make_async_copy(dst_hbm.at[pl.ds(ebase + kk * CH, CH)],
                                  didx[b], semd[b]).wait()
            pltpu.make_async_copy(x_hbm.at[sidx[b]], rows[b], semr[b]).wait()
            pltpu.async_copy(rows[b], aggr_sh.at[didx[b]], semsc[b], add=True)

            @pl.when(kk + 2 < CPW)
            def _():
                pltpu.async_copy(src_hbm.at[pl.ds(ebase + (kk + 2) * CH, CH)],
                                 sidx[b], sems[b])
        return carry

    lax.fori_loop(0, CPW // 2, step, 0)

    pltpu.make_async_copy(rows[0], aggr_sh.at[didx[0]], semsc[0]).wait()
    pltpu.make_async_copy(rows[1], aggr_sh.at[didx[1]], semsc[1]).wait()

    plsc.subcore_barrier()
    pltpu.sync_copy(aggr_sh.at[pl.ds(s * ROWS_PER_TILE, ROWS_PER_TILE)],
                    out_hbm.at[c, pl.ds(s * ROWS_PER_TILE, ROWS_PER_TILE)])


@functools.cache
def _sc_segment_sum():
    return pl.kernel(
        _sc_body,
        out_type=jax.ShapeDtypeStruct((NC, NP, D), jnp.float32),
        mesh=plsc.VectorSubcoreMesh(core_axis_name="c", subcore_axis_name="s",
                                    num_cores=NC, num_subcores=NS),
        scratch_types=[
            pltpu.VMEM((CH,), jnp.int32),
            pltpu.VMEM((CH,), jnp.int32),
            pltpu.VMEM((CH,), jnp.int32),
            pltpu.VMEM((CH,), jnp.int32),
            pltpu.VMEM((CH, D), jnp.float32),
            pltpu.VMEM((CH, D), jnp.float32),
            pltpu.VMEM_SHARED((NP, D), jnp.float32),
            pltpu.SemaphoreType.DMA,
            pltpu.SemaphoreType.DMA,
            pltpu.SemaphoreType.DMA,
            pltpu.SemaphoreType.DMA,
            pltpu.SemaphoreType.DMA,
            pltpu.SemaphoreType.DMA,
            pltpu.SemaphoreType.DMA,
            pltpu.SemaphoreType.DMA,
        ],
    )


ROW_BLK = 2000


def _tc_body(x_ref, parts_ref, w_ref, b_ref, s_ref, *out_refs):
    xa = jnp.concatenate([x_ref[...], parts_ref[0] + parts_ref[1]], axis=1)
    for level in range(NUM_LEVELS):
        xc = (jnp.dot(xa, w_ref[level], preferred_element_type=jnp.float32)
              + b_ref[level][None, :])
        big = jnp.dot(xc, s_ref[level], preferred_element_type=jnp.float32)
        for j in range(3):
            out_refs[level * 3 + j][...] = big[:, j * H:(j + 1) * H]


def _tc_dense(x, parts, Wstack, b, Sstack):
    grid = (N // ROW_BLK,)
    full = lambda shape: pl.BlockSpec(shape, lambda i: (0,) * len(shape))
    return pl.pallas_call(
        _tc_body,
        grid=grid,
        in_specs=[
            pl.BlockSpec((ROW_BLK, D), lambda i: (i, 0)),
            pl.BlockSpec((NC, ROW_BLK, D), lambda i: (0, i, 0)),
            full((NUM_LEVELS, 2 * D, H)),
            full((NUM_LEVELS, H)),
            full((NUM_LEVELS, H, 3 * H)),
        ],
        out_specs=tuple(pl.BlockSpec((ROW_BLK, H), lambda i: (i, 0))
                        for _ in range(NUM_PROC)),
        out_shape=tuple(jax.ShapeDtypeStruct((N, H), jnp.float32)
                        for _ in range(NUM_PROC)),
    )(x, parts, Wstack, b, Sstack)


def kernel(x, edge_index, Wroot, Wrel, b, S):
    pad = EPAD - E
    pad_ar = jnp.arange(pad, dtype=jnp.int32)
    src = jnp.concatenate([edge_index[0], pad_ar % N])
    dst = jnp.concatenate([edge_index[1], N + (pad_ar % (NP - N))])
    parts = _sc_segment_sum()(x, src, dst)
    Wstack = jnp.concatenate([Wroot, Wrel], axis=1)
    Sstack = jnp.transpose(S.reshape(NUM_LEVELS, 3, H, H),
                           (0, 2, 1, 3)).reshape(NUM_LEVELS, H, 3 * H)
    outs = _tc_dense(x, parts, Wstack, b, Sstack)
    return tuple(outs)
